# baseline (device time: 738309 ns/iter reference)
import jax
import jax.numpy as jnp
from jax import lax
from jax.experimental import pallas as pl
from jax.experimental.pallas import tpu as pltpu

NZ = 4
M, N = 16384, 1024
HALF = M // 2
R = 512
C = HALF // R


def _cast_kernel(x):
    def body(x_ref, o_ref):
        o_ref[...] = x_ref[...].astype(jnp.bfloat16)

    blk = 1024
    return pl.pallas_call(
        body,
        grid=(M // blk,),
        in_specs=[pl.BlockSpec((blk, N), lambda i: (i, 0))],
        out_specs=pl.BlockSpec((blk, N), lambda i: (i, 0)),
        out_shape=jax.ShapeDtypeStruct((M, N), jnp.bfloat16),
    )(x)


def _ar_kernel(xb):
    def body(x_ref, out_ref, acc, rbuf,
             load_sems, red_send, red_recv, bc_send, bc_recv,
             xs_send, xs_recv, out_sems):
        my_x = lax.axis_index("x")
        my_y = lax.axis_index("y")
        my_z = lax.axis_index("z")
        peer_x = 1 - my_x
        base = my_x * HALF
        obase = peer_x * HALF

        def ra(c):
            return pl.ds(c * R, R)

        def rm(c):
            return pl.ds(base + c * R, R)

        def send_chunk(src, dst, ssem, rsem, dz=0, to_xpeer=False):
            return pltpu.make_async_remote_copy(
                src_ref=src, dst_ref=dst, send_sem=ssem, recv_sem=rsem,
                device_id=(peer_x if to_xpeer else my_x, my_y, my_z + dz),
                device_id_type=pl.DeviceIdType.MESH,
            )

        barrier = pltpu.get_barrier_semaphore()

        @pl.when(my_z > 0)
        def _():
            pl.semaphore_signal(barrier, inc=1,
                                device_id=(my_x, my_y, my_z - 1),
                                device_id_type=pl.DeviceIdType.MESH)

        @pl.when(my_z < NZ - 1)
        def _():
            pl.semaphore_signal(barrier, inc=1,
                                device_id=(my_x, my_y, my_z + 1),
                                device_id_type=pl.DeviceIdType.MESH)

        pl.semaphore_signal(barrier, inc=1,
                            device_id=(peer_x, my_y, my_z),
                            device_id_type=pl.DeviceIdType.MESH)
        nnb = (1 + jnp.where(my_z > 0, 1, 0) + jnp.where(my_z < NZ - 1, 1, 0))
        pl.semaphore_wait(barrier, nnb)

        @pl.when(my_z > 0)
        def _():
            for c in range(C):
                pltpu.make_async_copy(
                    x_ref.at[rm(c), :], acc.at[ra(c), :], load_sems.at[c]
                ).start()

        LAG = 6
        for c in range(C + LAG):
            if c < C:
                @pl.when(my_z == 0)
                def _(c=c):
                    send_chunk(x_ref.at[rm(c), :], rbuf.at[ra(c), :],
                               red_send.at[c], red_recv.at[c], dz=+1).start()

                @pl.when(my_z > 0)
                def _(c=c):
                    pltpu.make_async_copy(
                        x_ref.at[rm(c), :], acc.at[ra(c), :], load_sems.at[c]
                    ).wait()
                    send_chunk(rbuf.at[ra(c), :], rbuf.at[ra(c), :],
                               red_send.at[c], red_recv.at[c]).wait_recv()
                    acc[ra(c), :] = acc[ra(c), :] + rbuf[ra(c), :]

                @pl.when((my_z > 0) & (my_z < NZ - 1))
                def _(c=c):
                    send_chunk(acc.at[ra(c), :], rbuf.at[ra(c), :],
                               red_send.at[c], red_recv.at[c], dz=+1).start()

                @pl.when(my_z == NZ - 1)
                def _(c=c):
                    send_chunk(acc.at[ra(c), :], out_ref.at[rm(c), :],
                               bc_send.at[c], bc_recv.at[c], dz=-1).start()
                    send_chunk(acc.at[ra(c), :], out_ref.at[rm(c), :],
                               xs_send.at[c], xs_recv.at[c],
                               to_xpeer=True).start()
                    pltpu.make_async_copy(
                        acc.at[ra(c), :], out_ref.at[rm(c), :], out_sems.at[c]
                    ).start()

            d = c - LAG
            if 0 <= d:
                @pl.when(my_z < NZ - 1)
                def _(d=d):
                    send_chunk(out_ref.at[rm(d), :], out_ref.at[rm(d), :],
                               bc_send.at[d], bc_recv.at[d]).wait_recv()

                @pl.when((my_z > 0) & (my_z < NZ - 1))
                def _(d=d):
                    send_chunk(out_ref.at[rm(d), :], out_ref.at[rm(d), :],
                               bc_send.at[d], bc_recv.at[d], dz=-1).start()

                @pl.when(my_z < NZ - 1)
                def _(d=d):
                    send_chunk(out_ref.at[rm(d), :], out_ref.at[rm(d), :],
                               xs_send.at[d], xs_recv.at[d],
                               to_xpeer=True).start()

        for c in range(C):
            ro = pl.ds(obase + c * R, R)
            send_chunk(out_ref.at[ro, :], out_ref.at[ro, :],
                       xs_send.at[c], xs_recv.at[c]).wait_recv()

            @pl.when(my_z < NZ - 1)
            def _(c=c):
                send_chunk(x_ref.at[rm(c), :], rbuf.at[ra(c), :],
                           red_send.at[c], red_recv.at[c]).wait_send()
                send_chunk(out_ref.at[rm(c), :], out_ref.at[rm(c), :],
                           xs_send.at[c], xs_recv.at[c]).wait_send()

            @pl.when(my_z > 0)
            def _(c=c):
                send_chunk(out_ref.at[rm(c), :], out_ref.at[rm(c), :],
                           bc_send.at[c], bc_recv.at[c]).wait_send()

            @pl.when(my_z == NZ - 1)
            def _(c=c):
                pltpu.make_async_copy(
                    acc.at[ra(c), :], out_ref.at[rm(c), :], out_sems.at[c]
                ).wait()
                send_chunk(acc.at[ra(c), :], out_ref.at[rm(c), :],
                           xs_send.at[c], xs_recv.at[c]).wait_send()

    return pl.pallas_call(
        body,
        out_shape=jax.ShapeDtypeStruct((M, N), jnp.bfloat16),
        in_specs=[pl.BlockSpec(memory_space=pl.ANY)],
        out_specs=pl.BlockSpec(memory_space=pl.ANY),
        scratch_shapes=[
            pltpu.VMEM((HALF, N), jnp.bfloat16),
            pltpu.VMEM((HALF, N), jnp.bfloat16),
            pltpu.SemaphoreType.DMA((C,)),
            pltpu.SemaphoreType.DMA((C,)),
            pltpu.SemaphoreType.DMA((C,)),
            pltpu.SemaphoreType.DMA((C,)),
            pltpu.SemaphoreType.DMA((C,)),
            pltpu.SemaphoreType.DMA((C,)),
            pltpu.SemaphoreType.DMA((C,)),
            pltpu.SemaphoreType.DMA((C,)),
        ],
        compiler_params=pltpu.CompilerParams(collective_id=0),
    )(xb)


def kernel(x):
    return _ar_kernel(_cast_kernel(x))


# device time: 508363 ns/iter; 1.4523x vs baseline; 1.4523x over previous
import jax
import jax.numpy as jnp
from jax import lax
from jax.experimental import pallas as pl
from jax.experimental.pallas import tpu as pltpu

NZ = 4
M, N = 16384, 1024
HALF = M // 2
QROWS = HALF // 2
R = 512
C = QROWS // R
QD = ((0, 1), (1, -1))


def _cast_kernel(x):
    def body(x_ref, o_ref):
        o_ref[...] = x_ref[...].astype(jnp.bfloat16)

    blk = 1024
    return pl.pallas_call(
        body,
        grid=(M // blk,),
        in_specs=[pl.BlockSpec((blk, N), lambda i: (i, 0))],
        out_specs=pl.BlockSpec((blk, N), lambda i: (i, 0)),
        out_shape=jax.ShapeDtypeStruct((M, N), jnp.bfloat16),
    )(x)


def _ar_kernel(x):
    def body(x_ref, out_ref, acc, rbuf,
             load_sems, red_send, red_recv, bc_send, bc_recv,
             xs_send, xs_recv, out_sems):
        my_x = lax.axis_index("x")
        my_y = lax.axis_index("y")
        my_z = lax.axis_index("z")
        peer_x = 1 - my_x
        base = my_x * HALF
        obase = peer_x * HALF

        def zeff(d):
            return my_z if d == 1 else NZ - 1 - my_z

        def rv(q, c):
            return pl.ds(q * QROWS + c * R, R)

        def rfull(q, c, b):
            return pl.ds(b + q * QROWS + c * R, R)

        def send_chunk(src, dst, ssem, rsem, dz=0, to_xpeer=False):
            return pltpu.make_async_remote_copy(
                src_ref=src, dst_ref=dst, send_sem=ssem, recv_sem=rsem,
                device_id=(peer_x if to_xpeer else my_x, my_y, my_z + dz),
                device_id_type=pl.DeviceIdType.MESH,
            )

        def load(q, c):
            return pltpu.make_async_copy(
                x_ref.at[rfull(q, c, base), :], acc.at[rv(q, c), :],
                load_sems.at[q, c])

        barrier = pltpu.get_barrier_semaphore()

        @pl.when(my_z > 0)
        def _():
            pl.semaphore_signal(barrier, inc=1,
                                device_id=(my_x, my_y, my_z - 1),
                                device_id_type=pl.DeviceIdType.MESH)

        @pl.when(my_z < NZ - 1)
        def _():
            pl.semaphore_signal(barrier, inc=1,
                                device_id=(my_x, my_y, my_z + 1),
                                device_id_type=pl.DeviceIdType.MESH)

        pl.semaphore_signal(barrier, inc=1,
                            device_id=(peer_x, my_y, my_z),
                            device_id_type=pl.DeviceIdType.MESH)
        nnb = (1 + jnp.where(my_z > 0, 1, 0) + jnp.where(my_z < NZ - 1, 1, 0))
        pl.semaphore_wait(barrier, nnb)

        for c in range(C):
            for q, d in QD:
                load(q, c).start()

        for c in range(C):
            for q, d in QD:
                @pl.when(zeff(d) == 0)
                def _(q=q, c=c, d=d):
                    load(q, c).wait()
                    send_chunk(acc.at[rv(q, c), :], rbuf.at[rv(q, c), :],
                               red_send.at[q, c], red_recv.at[q, c],
                               dz=d).start()

        for c in range(C):
            for q, d in QD:
                ze = zeff(d)

                @pl.when(ze > 0)
                def _(q=q, c=c, d=d):
                    load(q, c).wait()
                    send_chunk(rbuf.at[rv(q, c), :], rbuf.at[rv(q, c), :],
                               red_send.at[q, c],
                               red_recv.at[q, c]).wait_recv()
                    acc[rv(q, c), :] = acc[rv(q, c), :] + rbuf[rv(q, c), :]

                @pl.when((ze > 0) & (ze < NZ - 1))
                def _(q=q, c=c, d=d):
                    send_chunk(acc.at[rv(q, c), :], rbuf.at[rv(q, c), :],
                               red_send.at[q, c], red_recv.at[q, c],
                               dz=d).start()

                @pl.when(ze == NZ - 1)
                def _(q=q, c=c, d=d):
                    send_chunk(acc.at[rv(q, c), :],
                               out_ref.at[rfull(q, c, base), :],
                               bc_send.at[q, c], bc_recv.at[q, c],
                               dz=-d).start()
                    send_chunk(acc.at[rv(q, c), :],
                               out_ref.at[rfull(q, c, base), :],
                               xs_send.at[q, c], xs_recv.at[q, c],
                               to_xpeer=True).start()
                    pltpu.make_async_copy(
                        acc.at[rv(q, c), :],
                        out_ref.at[rfull(q, c, base), :],
                        out_sems.at[q, c]).start()

        for c in range(C):
            for q, d in QD:
                ze = zeff(d)
                rows = rfull(q, c, base)

                @pl.when(ze < NZ - 1)
                def _(q=q, c=c, rows=rows):
                    send_chunk(out_ref.at[rows, :], out_ref.at[rows, :],
                               bc_send.at[q, c],
                               bc_recv.at[q, c]).wait_recv()

                @pl.when((ze > 0) & (ze < NZ - 1))
                def _(q=q, c=c, d=d, rows=rows):
                    send_chunk(out_ref.at[rows, :], out_ref.at[rows, :],
                               bc_send.at[q, c], bc_recv.at[q, c],
                               dz=-d).start()

                @pl.when(ze < NZ - 1)
                def _(q=q, c=c, rows=rows):
                    send_chunk(out_ref.at[rows, :], out_ref.at[rows, :],
                               xs_send.at[q, c], xs_recv.at[q, c],
                               to_xpeer=True).start()

        for c in range(C):
            for q, d in QD:
                ze = zeff(d)
                rows = rfull(q, c, base)
                orows = rfull(q, c, obase)

                send_chunk(out_ref.at[orows, :], out_ref.at[orows, :],
                           xs_send.at[q, c], xs_recv.at[q, c]).wait_recv()
                send_chunk(out_ref.at[rows, :], out_ref.at[rows, :],
                           xs_send.at[q, c], xs_recv.at[q, c]).wait_send()

                @pl.when(ze < NZ - 1)
                def _(q=q, c=c, rows=rows):
                    send_chunk(out_ref.at[rows, :], rbuf.at[rv(q, c), :],
                               red_send.at[q, c],
                               red_recv.at[q, c]).wait_send()

                @pl.when(ze > 0)
                def _(q=q, c=c, rows=rows):
                    send_chunk(out_ref.at[rows, :], out_ref.at[rows, :],
                               bc_send.at[q, c],
                               bc_recv.at[q, c]).wait_send()

                @pl.when(ze == NZ - 1)
                def _(q=q, c=c, rows=rows):
                    pltpu.make_async_copy(
                        acc.at[rv(q, c), :], out_ref.at[rows, :],
                        out_sems.at[q, c]).wait()

    return pl.pallas_call(
        body,
        out_shape=jax.ShapeDtypeStruct((M, N), jnp.bfloat16),
        in_specs=[pl.BlockSpec(memory_space=pl.ANY)],
        out_specs=pl.BlockSpec(memory_space=pl.ANY),
        scratch_shapes=[
            pltpu.VMEM((HALF, N), jnp.bfloat16),
            pltpu.VMEM((HALF, N), jnp.bfloat16),
            pltpu.SemaphoreType.DMA((2, C)),
            pltpu.SemaphoreType.DMA((2, C)),
            pltpu.SemaphoreType.DMA((2, C)),
            pltpu.SemaphoreType.DMA((2, C)),
            pltpu.SemaphoreType.DMA((2, C)),
            pltpu.SemaphoreType.DMA((2, C)),
            pltpu.SemaphoreType.DMA((2, C)),
            pltpu.SemaphoreType.DMA((2, C)),
        ],
        compiler_params=pltpu.CompilerParams(collective_id=0),
    )(x)


def kernel(x):
    return _ar_kernel(_cast_kernel(x))


# device time: 500440 ns/iter; 1.4753x vs baseline; 1.0158x over previous
import jax
import jax.numpy as jnp
from jax import lax
from jax.experimental import pallas as pl
from jax.experimental.pallas import tpu as pltpu

NZ = 4
M, N = 16384, 1024
HALF = M // 2
QROWS = HALF // 2
R = 1024
C = QROWS // R
QD = ((0, 1), (1, -1))


def _cast_kernel(x):
    def body(x_ref, o_ref):
        o_ref[...] = x_ref[...].astype(jnp.bfloat16)

    blk = 1024
    return pl.pallas_call(
        body,
        grid=(M // blk,),
        in_specs=[pl.BlockSpec((blk, N), lambda i: (i, 0))],
        out_specs=pl.BlockSpec((blk, N), lambda i: (i, 0)),
        out_shape=jax.ShapeDtypeStruct((M, N), jnp.bfloat16),
    )(x)


def _ar_kernel(x):
    def body(x_ref, out_ref, acc, rbuf,
             load_sems, red_send, red_recv, bc_send, bc_recv,
             xs_send, xs_recv, out_sems):
        my_x = lax.axis_index("x")
        my_y = lax.axis_index("y")
        my_z = lax.axis_index("z")
        peer_x = 1 - my_x
        base = my_x * HALF
        obase = peer_x * HALF

        def zeff(d):
            return my_z if d == 1 else NZ - 1 - my_z

        def rv(q, c):
            return pl.ds(q * QROWS + c * R, R)

        def rfull(q, c, b):
            return pl.ds(b + q * QROWS + c * R, R)

        def send_chunk(src, dst, ssem, rsem, dz=0, to_xpeer=False):
            return pltpu.make_async_remote_copy(
                src_ref=src, dst_ref=dst, send_sem=ssem, recv_sem=rsem,
                device_id=(peer_x if to_xpeer else my_x, my_y, my_z + dz),
                device_id_type=pl.DeviceIdType.MESH,
            )

        def load(q, c):
            return pltpu.make_async_copy(
                x_ref.at[rfull(q, c, base), :], acc.at[rv(q, c), :],
                load_sems.at[q, c])

        barrier = pltpu.get_barrier_semaphore()

        @pl.when(my_z > 0)
        def _():
            pl.semaphore_signal(barrier, inc=1,
                                device_id=(my_x, my_y, my_z - 1),
                                device_id_type=pl.DeviceIdType.MESH)

        @pl.when(my_z < NZ - 1)
        def _():
            pl.semaphore_signal(barrier, inc=1,
                                device_id=(my_x, my_y, my_z + 1),
                                device_id_type=pl.DeviceIdType.MESH)

        pl.semaphore_signal(barrier, inc=1,
                            device_id=(peer_x, my_y, my_z),
                            device_id_type=pl.DeviceIdType.MESH)
        nnb = (1 + jnp.where(my_z > 0, 1, 0) + jnp.where(my_z < NZ - 1, 1, 0))
        pl.semaphore_wait(barrier, nnb)

        for c in range(C):
            for q, d in QD:
                load(q, c).start()

        for c in range(C):
            for q, d in QD:
                @pl.when(zeff(d) == 0)
                def _(q=q, c=c, d=d):
                    load(q, c).wait()
                    send_chunk(acc.at[rv(q, c), :], rbuf.at[rv(q, c), :],
                               red_send.at[q, c], red_recv.at[q, c],
                               dz=d).start()

        for c in range(C):
            for q, d in QD:
                ze = zeff(d)

                @pl.when(ze > 0)
                def _(q=q, c=c, d=d):
                    load(q, c).wait()
                    send_chunk(rbuf.at[rv(q, c), :], rbuf.at[rv(q, c), :],
                               red_send.at[q, c],
                               red_recv.at[q, c]).wait_recv()
                    acc[rv(q, c), :] = acc[rv(q, c), :] + rbuf[rv(q, c), :]

                @pl.when((ze > 0) & (ze < NZ - 1))
                def _(q=q, c=c, d=d):
                    send_chunk(acc.at[rv(q, c), :], rbuf.at[rv(q, c), :],
                               red_send.at[q, c], red_recv.at[q, c],
                               dz=d).start()

                @pl.when(ze == NZ - 1)
                def _(q=q, c=c, d=d):
                    send_chunk(acc.at[rv(q, c), :],
                               out_ref.at[rfull(q, c, base), :],
                               bc_send.at[q, c], bc_recv.at[q, c],
                               dz=-d).start()
                    send_chunk(acc.at[rv(q, c), :],
                               out_ref.at[rfull(q, c, base), :],
                               xs_send.at[q, c], xs_recv.at[q, c],
                               to_xpeer=True).start()
                    pltpu.make_async_copy(
                        acc.at[rv(q, c), :],
                        out_ref.at[rfull(q, c, base), :],
                        out_sems.at[q, c]).start()

        for c in range(C):
            for q, d in QD:
                ze = zeff(d)
                rows = rfull(q, c, base)

                @pl.when(ze < NZ - 1)
                def _(q=q, c=c, rows=rows):
                    send_chunk(out_ref.at[rows, :], out_ref.at[rows, :],
                               bc_send.at[q, c],
                               bc_recv.at[q, c]).wait_recv()

                @pl.when((ze > 0) & (ze < NZ - 1))
                def _(q=q, c=c, d=d, rows=rows):
                    send_chunk(out_ref.at[rows, :], out_ref.at[rows, :],
                               bc_send.at[q, c], bc_recv.at[q, c],
                               dz=-d).start()

                @pl.when(ze < NZ - 1)
                def _(q=q, c=c, rows=rows):
                    send_chunk(out_ref.at[rows, :], out_ref.at[rows, :],
                               xs_send.at[q, c], xs_recv.at[q, c],
                               to_xpeer=True).start()

        for c in range(C):
            for q, d in QD:
                ze = zeff(d)
                rows = rfull(q, c, base)
                orows = rfull(q, c, obase)

                send_chunk(out_ref.at[orows, :], out_ref.at[orows, :],
                           xs_send.at[q, c], xs_recv.at[q, c]).wait_recv()
                send_chunk(out_ref.at[rows, :], out_ref.at[rows, :],
                           xs_send.at[q, c], xs_recv.at[q, c]).wait_send()

                @pl.when(ze < NZ - 1)
                def _(q=q, c=c, rows=rows):
                    send_chunk(out_ref.at[rows, :], rbuf.at[rv(q, c), :],
                               red_send.at[q, c],
                               red_recv.at[q, c]).wait_send()

                @pl.when(ze > 0)
                def _(q=q, c=c, rows=rows):
                    send_chunk(out_ref.at[rows, :], out_ref.at[rows, :],
                               bc_send.at[q, c],
                               bc_recv.at[q, c]).wait_send()

                @pl.when(ze == NZ - 1)
                def _(q=q, c=c, rows=rows):
                    pltpu.make_async_copy(
                        acc.at[rv(q, c), :], out_ref.at[rows, :],
                        out_sems.at[q, c]).wait()

    return pl.pallas_call(
        body,
        out_shape=jax.ShapeDtypeStruct((M, N), jnp.bfloat16),
        in_specs=[pl.BlockSpec(memory_space=pl.ANY)],
        out_specs=pl.BlockSpec(memory_space=pl.ANY),
        scratch_shapes=[
            pltpu.VMEM((HALF, N), jnp.bfloat16),
            pltpu.VMEM((HALF, N), jnp.bfloat16),
            pltpu.SemaphoreType.DMA((2, C)),
            pltpu.SemaphoreType.DMA((2, C)),
            pltpu.SemaphoreType.DMA((2, C)),
            pltpu.SemaphoreType.DMA((2, C)),
            pltpu.SemaphoreType.DMA((2, C)),
            pltpu.SemaphoreType.DMA((2, C)),
            pltpu.SemaphoreType.DMA((2, C)),
            pltpu.SemaphoreType.DMA((2, C)),
        ],
        compiler_params=pltpu.CompilerParams(collective_id=0),
    )(x)


def kernel(x):
    return _ar_kernel(_cast_kernel(x))


# device time: 376894 ns/iter; 1.9589x vs baseline; 1.3278x over previous
import jax
import jax.numpy as jnp
from jax import lax
from jax.experimental import pallas as pl
from jax.experimental.pallas import tpu as pltpu

NZ = 4
NY = 4
M, N = 16384, 1024
HALF = M // 2
SEG = HALF // NY
R = 512
C = SEG // R


def _cast_kernel(x):
    def body(x_ref, o_ref):
        o_ref[...] = x_ref[...].astype(jnp.bfloat16)

    blk = 1024
    return pl.pallas_call(
        body,
        grid=(M // blk,),
        in_specs=[pl.BlockSpec((blk, N), lambda i: (i, 0))],
        out_specs=pl.BlockSpec((blk, N), lambda i: (i, 0)),
        out_shape=jax.ShapeDtypeStruct((M, N), jnp.bfloat16),
    )(x)


def _ar_kernel(x):
    def body(x_ref, out_ref, acc, rbuf,
             load_sems, red_send, red_recv, bc_send, bc_recv,
             y_send, y_recv, xs_send, xs_recv, out_sems):
        my_x = lax.axis_index("x")
        my_y = lax.axis_index("y")
        my_z = lax.axis_index("z")
        peer_x = 1 - my_x
        base = my_x * HALF
        obase = peer_x * HALF
        seg_base = base + my_y * SEG

        def rv(c):
            return pl.ds(c * R, R)

        def rmine(c):
            return pl.ds(seg_base + c * R, R)

        def rseg(b, j, c):
            return pl.ds(b + j * SEG + c * R, R)

        def send_chunk(src, dst, ssem, rsem, dz=0, dy=0, to_xpeer=False):
            return pltpu.make_async_remote_copy(
                src_ref=src, dst_ref=dst, send_sem=ssem, recv_sem=rsem,
                device_id=(peer_x if to_xpeer else my_x,
                           (my_y + dy) % NY, my_z + dz),
                device_id_type=pl.DeviceIdType.MESH,
            )

        def load(c):
            return pltpu.make_async_copy(
                x_ref.at[rmine(c), :], acc.at[rv(c), :], load_sems.at[c])

        barrier = pltpu.get_barrier_semaphore()

        @pl.when(my_z > 0)
        def _():
            pl.semaphore_signal(barrier, inc=1,
                                device_id=(my_x, my_y, my_z - 1),
                                device_id_type=pl.DeviceIdType.MESH)

        @pl.when(my_z < NZ - 1)
        def _():
            pl.semaphore_signal(barrier, inc=1,
                                device_id=(my_x, my_y, my_z + 1),
                                device_id_type=pl.DeviceIdType.MESH)

        pl.semaphore_signal(barrier, inc=1,
                            device_id=(peer_x, my_y, my_z),
                            device_id_type=pl.DeviceIdType.MESH)
        for dy in (1, 2, 3):
            pl.semaphore_signal(barrier, inc=1,
                                device_id=(my_x, (my_y + dy) % NY, my_z),
                                device_id_type=pl.DeviceIdType.MESH)
        nnb = (4 + jnp.where(my_z > 0, 1, 0) + jnp.where(my_z < NZ - 1, 1, 0))
        pl.semaphore_wait(barrier, nnb)

        for c in range(C):
            load(c).start()

        def fanout(c, src):
            for dy in (1, 2, 3):
                send_chunk(src, out_ref.at[rmine(c), :],
                           y_send.at[dy - 1, c], y_recv.at[dy - 1, c],
                           dy=dy).start()
            send_chunk(src, out_ref.at[rmine(c), :],
                       xs_send.at[0, c], xs_recv.at[0, c],
                       to_xpeer=True).start()

        for c in range(C):
            @pl.when(my_z == 0)
            def _(c=c):
                load(c).wait()
                send_chunk(acc.at[rv(c), :], rbuf.at[rv(c), :],
                           red_send.at[c], red_recv.at[c], dz=+1).start()

            @pl.when(my_z > 0)
            def _(c=c):
                load(c).wait()
                send_chunk(rbuf.at[rv(c), :], rbuf.at[rv(c), :],
                           red_send.at[c], red_recv.at[c]).wait_recv()
                acc[rv(c), :] = acc[rv(c), :] + rbuf[rv(c), :]

            @pl.when((my_z > 0) & (my_z < NZ - 1))
            def _(c=c):
                send_chunk(acc.at[rv(c), :], rbuf.at[rv(c), :],
                           red_send.at[c], red_recv.at[c], dz=+1).start()

            @pl.when(my_z == NZ - 1)
            def _(c=c):
                send_chunk(acc.at[rv(c), :], out_ref.at[rmine(c), :],
                           bc_send.at[c], bc_recv.at[c], dz=-1).start()
                pltpu.make_async_copy(
                    acc.at[rv(c), :], out_ref.at[rmine(c), :],
                    out_sems.at[c]).start()
                fanout(c, acc.at[rv(c), :])

        for c in range(C):
            @pl.when(my_z < NZ - 1)
            def _(c=c):
                send_chunk(out_ref.at[rmine(c), :], out_ref.at[rmine(c), :],
                           bc_send.at[c], bc_recv.at[c]).wait_recv()

            @pl.when((my_z > 0) & (my_z < NZ - 1))
            def _(c=c):
                send_chunk(out_ref.at[rmine(c), :], out_ref.at[rmine(c), :],
                           bc_send.at[c], bc_recv.at[c], dz=-1).start()

            @pl.when(my_z < NZ - 1)
            def _(c=c):
                fanout(c, out_ref.at[rmine(c), :])

        for dy in (1, 2, 3):
            j = (my_y - dy) % NY
            for c in range(C):
                rj = rseg(base, j, c)
                send_chunk(out_ref.at[rj, :], out_ref.at[rj, :],
                           y_send.at[dy - 1, c],
                           y_recv.at[dy - 1, c]).wait_recv()
                send_chunk(out_ref.at[rj, :], out_ref.at[rj, :],
                           xs_send.at[dy, c], xs_recv.at[dy, c],
                           to_xpeer=True).start()

        for dy in range(4):
            j = (my_y - dy) % NY
            for c in range(C):
                ro = rseg(obase, j, c)
                send_chunk(out_ref.at[ro, :], out_ref.at[ro, :],
                           xs_send.at[dy, c], xs_recv.at[dy, c]).wait_recv()
                send_chunk(out_ref.at[rseg(base, j, c), :],
                           out_ref.at[rseg(base, j, c), :],
                           xs_send.at[dy, c], xs_recv.at[dy, c]).wait_send()

        for c in range(C):
            for dy in (1, 2, 3):
                send_chunk(out_ref.at[rmine(c), :], out_ref.at[rmine(c), :],
                           y_send.at[dy - 1, c],
                           y_recv.at[dy - 1, c]).wait_send()

            @pl.when(my_z < NZ - 1)
            def _(c=c):
                send_chunk(out_ref.at[rmine(c), :], rbuf.at[rv(c), :],
                           red_send.at[c], red_recv.at[c]).wait_send()

            @pl.when(my_z > 0)
            def _(c=c):
                send_chunk(out_ref.at[rmine(c), :], out_ref.at[rmine(c), :],
                           bc_send.at[c], bc_recv.at[c]).wait_send()

            @pl.when(my_z == NZ - 1)
            def _(c=c):
                pltpu.make_async_copy(
                    acc.at[rv(c), :], out_ref.at[rmine(c), :],
                    out_sems.at[c]).wait()

    return pl.pallas_call(
        body,
        out_shape=jax.ShapeDtypeStruct((M, N), jnp.bfloat16),
        in_specs=[pl.BlockSpec(memory_space=pl.ANY)],
        out_specs=pl.BlockSpec(memory_space=pl.ANY),
        scratch_shapes=[
            pltpu.VMEM((SEG, N), jnp.bfloat16),
            pltpu.VMEM((SEG, N), jnp.bfloat16),
            pltpu.SemaphoreType.DMA((C,)),
            pltpu.SemaphoreType.DMA((C,)),
            pltpu.SemaphoreType.DMA((C,)),
            pltpu.SemaphoreType.DMA((C,)),
            pltpu.SemaphoreType.DMA((C,)),
            pltpu.SemaphoreType.DMA((3, C)),
            pltpu.SemaphoreType.DMA((3, C)),
            pltpu.SemaphoreType.DMA((4, C)),
            pltpu.SemaphoreType.DMA((4, C)),
            pltpu.SemaphoreType.DMA((C,)),
        ],
        compiler_params=pltpu.CompilerParams(collective_id=0),
    )(x)


def kernel(x):
    return _ar_kernel(_cast_kernel(x))
